# Initial kernel scaffold; baseline (speedup 1.0000x reference)
import jax
import jax.numpy as jnp
from jax.experimental import pallas as pl


def kernel(x, edge_index, batch, graph_feat, W1, b1, W2, b2, gW, gb, bn1_g, bn1_b, bn1_m, bn1_v, gfW, gfb, fbn_g, fbn_b, fbn_m, fbn_v, oW, ob):
    raise NotImplementedError("write your pallas kernel here")


# speedup vs baseline: 6.7921x; 6.7921x over previous
import functools

import jax
import jax.numpy as jnp
from jax import lax
from jax.experimental import pallas as pl
from jax.experimental.pallas import tpu as pltpu
from jax.experimental.pallas import tpu_sc as plsc

N = 10000
E = 320000
D = 128
H = 256
G = 64
GF = 32
C = 10

NC = 2
NS = 16
NW = NC * NS
EP = E // NW
CH = 128
NFULL = EP // CH
REM = EP - NFULL * CH
ROWS_PT = 624
ROWS_LAST = N - (NS - 1) * ROWS_PT

@functools.cache
def _sc_scatter_add_fn():
    mesh = plsc.VectorSubcoreMesh(core_axis_name="c", subcore_axis_name="s",
                                  num_cores=NC, num_subcores=NS)
    return functools.partial(
        pl.kernel,
        mesh=mesh,
        out_type=jax.ShapeDtypeStruct((NC * N, D), jnp.float32),
        scratch_types=[
            pltpu.VMEM((CH,), jnp.int32),
            pltpu.VMEM((CH,), jnp.int32),
            pltpu.VMEM((CH, D), jnp.float32),
            pltpu.VMEM((REM,), jnp.int32),
            pltpu.VMEM((REM,), jnp.int32),
            pltpu.VMEM((REM, D), jnp.float32),
            pltpu.VMEM_SHARED((N, D), jnp.float32),
            pltpu.SemaphoreType.DMA,
        ],
    )(_sc_scatter_add_body)


def _sc_scatter_add_body(x_hbm, src_hbm, dst_hbm, zeros_hbm, out_hbm,
                         sidx, didx, rows, sidx_r, didx_r, rows_r, agg_sh, sem):
    c = lax.axis_index("c")
    s = lax.axis_index("s")
    t = c * NS + s
    row0 = pl.multiple_of(s * ROWS_PT, 8)
    out0 = pl.multiple_of(c * N + s * ROWS_PT, 8)
    last = NS - 1

    @pl.when(s != last)
    def _():
        pltpu.sync_copy(zeros_hbm.at[pl.ds(0, ROWS_PT)],
                        agg_sh.at[pl.ds(row0, ROWS_PT)])

    @pl.when(s == last)
    def _():
        pltpu.sync_copy(zeros_hbm, agg_sh.at[pl.ds(row0, ROWS_LAST)])

    plsc.subcore_barrier()

    ebase = pl.multiple_of(t * EP, 8)

    def body(k, carry):
        base = pl.multiple_of(ebase + k * CH, 8)
        pltpu.sync_copy(src_hbm.at[pl.ds(base, CH)], sidx)
        pltpu.sync_copy(dst_hbm.at[pl.ds(base, CH)], didx)
        pltpu.async_copy(x_hbm.at[sidx], rows, sem).wait()
        pltpu.sync_copy(rows, agg_sh.at[didx], add=True)
        return carry

    lax.fori_loop(0, NFULL, body, 0)

    rbase = pl.multiple_of(ebase + NFULL * CH, 8)
    pltpu.sync_copy(src_hbm.at[pl.ds(rbase, REM)], sidx_r)
    pltpu.sync_copy(dst_hbm.at[pl.ds(rbase, REM)], didx_r)
    pltpu.async_copy(x_hbm.at[sidx_r], rows_r, sem).wait()
    pltpu.sync_copy(rows_r, agg_sh.at[didx_r], add=True)

    plsc.subcore_barrier()

    @pl.when(s != last)
    def _():
        pltpu.sync_copy(agg_sh.at[pl.ds(row0, ROWS_PT)],
                        out_hbm.at[pl.ds(out0, ROWS_PT)])

    @pl.when(s == last)
    def _():
        pltpu.sync_copy(agg_sh.at[pl.ds(row0, ROWS_LAST)],
                        out_hbm.at[pl.ds(out0, ROWS_LAST)])


def _tc_body(x_ref, part_ref, batch_ref, gfeat_ref,
             W1_ref, b1_ref, W2_ref, b2_ref, gW_ref, gb_ref,
             bn_g_ref, bn_b_ref, bn_m_ref, bn_v_ref,
             gfW_ref, gfb_ref,
             fa_g_ref, fa_b_ref, fa_m_ref, fa_v_ref,
             fb_g_ref, fb_b_ref, fb_m_ref, fb_v_ref,
             oWa_ref, oWb_ref, ob_ref, out_ref):
    x = x_ref[...]
    agg = part_ref[:N, :] + part_ref[N:, :]
    h0 = x + agg
    h1 = jnp.maximum(
        jnp.dot(h0, W1_ref[...], preferred_element_type=jnp.float32)
        + b1_ref[...], 0.0)
    h = jnp.maximum(
        jnp.dot(h1, W2_ref[...], preferred_element_type=jnp.float32)
        + b2_ref[...], 0.0)
    gate = (jnp.dot(h, gW_ref[...], preferred_element_type=jnp.float32)
            + gb_ref[...])

    onehot = batch_ref[...] == lax.broadcasted_iota(jnp.int32, (1, G), 1)
    neg_inf = jnp.float32(-jnp.inf)
    masked = jnp.where(onehot, gate, neg_inf)
    gmax = jnp.max(masked, axis=0, keepdims=True)
    gmax = jnp.where(gmax == neg_inf, 0.0, gmax)
    gmax_n = jnp.sum(jnp.where(onehot, gmax, 0.0), axis=1, keepdims=True)
    e = jnp.exp(gate - gmax_n)
    denom = jnp.sum(jnp.where(onehot, e, 0.0), axis=0, keepdims=True)
    denom_n = jnp.sum(jnp.where(onehot, denom, 0.0), axis=1, keepdims=True)
    alpha = e / (denom_n + 1e-16)
    w = jnp.where(onehot, alpha, 0.0)
    pooled = lax.dot_general(w, h, (((0,), (0,)), ((), ())),
                             preferred_element_type=jnp.float32)
    pooled = ((pooled - bn_m_ref[...]) / jnp.sqrt(bn_v_ref[...] + 1e-5)
              * bn_g_ref[...] + bn_b_ref[...])
    gf = jnp.maximum(
        jnp.dot(gfeat_ref[...], gfW_ref[...],
                preferred_element_type=jnp.float32) + gfb_ref[...], 0.0)
    fa = ((pooled - fa_m_ref[...]) / jnp.sqrt(fa_v_ref[...] + 1e-5)
          * fa_g_ref[...] + fa_b_ref[...])
    fb = ((gf - fb_m_ref[...]) / jnp.sqrt(fb_v_ref[...] + 1e-5)
          * fb_g_ref[...] + fb_b_ref[...])
    logits = (jnp.dot(fa, oWa_ref[...], preferred_element_type=jnp.float32)
              + jnp.dot(fb, oWb_ref[...], preferred_element_type=jnp.float32)
              + ob_ref[...])
    m = jnp.max(logits, axis=1, keepdims=True)
    lse = jnp.log(jnp.sum(jnp.exp(logits - m), axis=1, keepdims=True)) + m
    out_ref[...] = logits - lse


def kernel(x, edge_index, batch, graph_feat, W1, b1, W2, b2, gW, gb,
           bn1_g, bn1_b, bn1_m, bn1_v, gfW, gfb, fbn_g, fbn_b, fbn_m, fbn_v,
           oW, ob):
    src = edge_index[0]
    dst = edge_index[1]
    zeros = jnp.zeros((ROWS_LAST, D), jnp.float32)
    part = _sc_scatter_add_fn()(x, src, dst, zeros)

    batch2 = batch.reshape(N, 1)
    row = lambda v: v.reshape(1, -1)
    out = pl.pallas_call(
        _tc_body,
        out_shape=jax.ShapeDtypeStruct((G, C), jnp.float32),
    )(x, part, batch2, graph_feat,
      W1, row(b1), W2, row(b2), gW, row(gb),
      row(bn1_g), row(bn1_b), row(bn1_m), row(bn1_v),
      gfW, row(gfb),
      row(fbn_g[:H]), row(fbn_b[:H]), row(fbn_m[:H]), row(fbn_v[:H]),
      row(fbn_g[H:]), row(fbn_b[H:]), row(fbn_m[H:]), row(fbn_v[H:]),
      oW[:H], oW[H:], row(ob))
    return out
